# Initial kernel scaffold; baseline (speedup 1.0000x reference)
import jax
import jax.numpy as jnp
from jax.experimental import pallas as pl


def kernel(x, tables):
    raise NotImplementedError("write your pallas kernel here")


# speedup vs baseline: 4.1638x; 4.1638x over previous
import functools

import jax
import jax.numpy as jnp
from jax import lax
from jax.experimental import pallas as pl
from jax.experimental.pallas import tpu as pltpu
from jax.experimental.pallas import tpu_sc as plsc

NUM_FEATURES = 21
VOCAB = 2000
EMB_DIM = 128
LANES = 16
NUM_CORES = 2
NUM_SUBCORES = 16
NUM_WORKERS = NUM_CORES * NUM_SUBCORES
B_STEP = 16
VREGS_PER_ROW = EMB_DIM // LANES


def _make_sc_call(n_pad):
    rows_per_tile = n_pad // NUM_WORKERS
    steps = rows_per_tile // B_STEP
    mesh = plsc.VectorSubcoreMesh(core_axis_name="c", subcore_axis_name="s")

    @functools.partial(
        pl.kernel,
        out_type=jax.ShapeDtypeStruct((n_pad, EMB_DIM), jnp.float32),
        mesh=mesh,
        scratch_types=[
            pltpu.VMEM((B_STEP * NUM_FEATURES,), jnp.int32),
            pltpu.VMEM((NUM_FEATURES, B_STEP, EMB_DIM), jnp.float32),
            pltpu.VMEM((B_STEP, EMB_DIM), jnp.float32),
            pltpu.SemaphoreType.DMA,
            pltpu.SemaphoreType.DMA,
        ],
        compiler_params=pltpu.CompilerParams(needs_layout_passes=False),
    )
    def sc_kernel(x_hbm, tab_hbm, out_hbm, xbuf, gbuf, obuf, gsem, osem):
        wid = lax.axis_index("s") * NUM_CORES + lax.axis_index("c")
        tile_base = wid * rows_per_tile
        row_ids = lax.iota(jnp.int32, LANES)

        def step_body(s, carry):
            base = tile_base + s * B_STEP
            pltpu.sync_copy(
                x_hbm.at[pl.ds(base * NUM_FEATURES, B_STEP * NUM_FEATURES)], xbuf
            )
            copies = []
            for i in range(NUM_FEATURES):
                flat_ids = row_ids * NUM_FEATURES + i
                idx = plsc.load_gather(xbuf, [flat_ids]) + (i * VOCAB)
                copies.append(pltpu.async_copy(tab_hbm.at[idx], gbuf.at[i], gsem))
            for c in copies:
                c.wait()

            def acc_body(j, _):
                for k in range(VREGS_PER_ROW):
                    sl = pl.ds(k * LANES, LANES)
                    acc = gbuf[0, j, sl]
                    for i in range(1, NUM_FEATURES):
                        acc = acc + gbuf[i, j, sl]
                    obuf[j, sl] = acc
                return 0

            lax.fori_loop(0, B_STEP, acc_body, 0, unroll=False)
            pltpu.async_copy(obuf, out_hbm.at[pl.ds(base, B_STEP), :], osem).wait()
            return carry

        lax.fori_loop(0, steps, step_body, 0, unroll=False)

    return sc_kernel


def kernel(x, tables):
    n = x.shape[0]
    block = NUM_WORKERS * B_STEP
    n_pad = ((n + block - 1) // block) * block
    if n_pad != n:
        x = jnp.pad(x, ((0, n_pad - n), (0, 0)))
    tab_flat = tables.reshape(NUM_FEATURES * VOCAB, EMB_DIM)
    out = _make_sc_call(n_pad)(x.reshape(-1), tab_flat)
    return out[:n]


# speedup vs baseline: 6.8632x; 1.6483x over previous
import functools

import jax
import jax.numpy as jnp
from jax import lax
from jax.experimental import pallas as pl
from jax.experimental.pallas import tpu as pltpu
from jax.experimental.pallas import tpu_sc as plsc

NUM_FEATURES = 21
VOCAB = 2000
EMB_DIM = 128
LANES = 16
NUM_CORES = 2
NUM_SUBCORES = 16
NUM_WORKERS = NUM_CORES * NUM_SUBCORES
B_STEP = 16
VREGS_PER_ROW = EMB_DIM // LANES


def _make_sc_call(n_pad):
    rows_per_tile = n_pad // NUM_WORKERS
    steps = rows_per_tile // B_STEP
    mesh = plsc.VectorSubcoreMesh(core_axis_name="c", subcore_axis_name="s")

    @functools.partial(
        pl.kernel,
        out_type=jax.ShapeDtypeStruct((n_pad, EMB_DIM), jnp.float32),
        mesh=mesh,
        scratch_types=[
            pltpu.VMEM((B_STEP * NUM_FEATURES,), jnp.int32),
            pltpu.VMEM((2, NUM_FEATURES * B_STEP, EMB_DIM), jnp.float32),
            pltpu.VMEM((2, B_STEP, EMB_DIM), jnp.float32),
            pltpu.SemaphoreType.DMA,
            pltpu.SemaphoreType.DMA,
            pltpu.SemaphoreType.DMA,
        ],
        compiler_params=pltpu.CompilerParams(needs_layout_passes=False),
    )
    def sc_kernel(x_hbm, tab_hbm, out_hbm, xbuf, gbuf, obuf, gsem0, gsem1, osem):
        wid = lax.axis_index("s") * NUM_CORES + lax.axis_index("c")
        tile_base = wid * rows_per_tile
        row_ids = lax.iota(jnp.int32, LANES)
        gsems = (gsem0, gsem1)
        gather_bytes = NUM_FEATURES * B_STEP * EMB_DIM * 4

        def fire(s, b):
            base = tile_base + s * B_STEP
            pltpu.sync_copy(
                x_hbm.at[pl.ds(base * NUM_FEATURES, B_STEP * NUM_FEATURES)], xbuf
            )
            for i in range(NUM_FEATURES):
                flat_ids = row_ids * NUM_FEATURES + i
                idx = plsc.load_gather(xbuf, [flat_ids]) + (i * VOCAB)
                pltpu.async_copy(
                    tab_hbm.at[idx], gbuf.at[b, pl.ds(i * B_STEP, B_STEP), :], gsems[b]
                )

        def drain_gathers(b):
            pltpu.make_async_copy(
                tab_hbm.at[pl.ds(0, NUM_FEATURES * B_STEP), :],
                gbuf.at[b],
                gsems[b],
            ).wait()

        def consume(s, b):
            base = tile_base + s * B_STEP
            drain_gathers(b)
            @pl.when(s >= 2)
            def _():
                pltpu.make_async_copy(
                    obuf.at[b], out_hbm.at[pl.ds(base, B_STEP), :], osem
                ).wait()

            def acc_body(j, _):
                for k in range(VREGS_PER_ROW):
                    sl = pl.ds(k * LANES, LANES)
                    acc = gbuf[b, j, sl]
                    for i in range(1, NUM_FEATURES):
                        acc = acc + gbuf[b, i * B_STEP + j, sl]
                    obuf[b, j, sl] = acc
                return 0

            lax.fori_loop(0, B_STEP, acc_body, 0, unroll=False)
            pltpu.async_copy(obuf.at[b], out_hbm.at[pl.ds(base, B_STEP), :], osem)

        fire(0, 0)

        def pair_body(it, carry):
            s = it * 2

            @pl.when(s + 1 < steps)
            def _():
                fire(s + 1, 1)

            consume(s, 0)

            @pl.when(s + 2 < steps)
            def _():
                fire(s + 2, 0)

            @pl.when(s + 1 < steps)
            def _():
                consume(s + 1, 1)

            return carry

        lax.fori_loop(0, (steps + 1) // 2, pair_body, 0, unroll=False)
        for b in ((steps - 2) % 2, (steps - 1) % 2):
            pltpu.make_async_copy(
                obuf.at[b], out_hbm.at[pl.ds(0, B_STEP), :], osem
            ).wait()

    return sc_kernel


def kernel(x, tables):
    n = x.shape[0]
    block = NUM_WORKERS * B_STEP
    n_pad = ((n + block - 1) // block) * block
    if n_pad != n:
        x = jnp.pad(x, ((0, n_pad - n), (0, 0)))
    tab_flat = tables.reshape(NUM_FEATURES * VOCAB, EMB_DIM)
    out = _make_sc_call(n_pad)(x.reshape(-1), tab_flat)
    return out[:n]
